# baseline (device time: 36561 ns/iter reference)
import jax
import jax.numpy as jnp
from jax import lax
from jax.experimental import pallas as pl
from jax.experimental.pallas import tpu as pltpu

N = 16
H = 2


def kernel(x):
    m, n = x.shape
    ch = m // N
    hh = ch // H

    def body(x_ref, out_ref, rs_buf, rs_send, rs_recv, ag_send, ag_recv):
        p = lax.axis_index("i")

        rs_rdmas = []
        for h in range(H):
            for d in range(1, N):
                t = (p + d) % N
                rdma = pltpu.make_async_remote_copy(
                    src_ref=x_ref.at[pl.ds(t * ch + h * hh, hh), :],
                    dst_ref=rs_buf.at[h, d],
                    send_sem=rs_send.at[h, d],
                    recv_sem=rs_recv.at[h, d],
                    device_id=(t,),
                    device_id_type=pl.DeviceIdType.MESH,
                )
                rdma.start()
                rs_rdmas.append(rdma)

        for h in range(H):
            rs_buf[h, 0, :, :] = x_ref[pl.ds(p * ch + h * hh, hh), :]

        ag_rdmas = []
        for h in range(H):
            for d in range(1, N):
                rs_rdmas[h * (N - 1) + d - 1].wait_recv()
            out_ref[pl.ds(p * ch + h * hh, hh), :] = jnp.sum(
                rs_buf[h], axis=0
            )
            for d in range(1, N):
                t = (p + d) % N
                rdma = pltpu.make_async_remote_copy(
                    src_ref=out_ref.at[pl.ds(p * ch + h * hh, hh), :],
                    dst_ref=out_ref.at[pl.ds(p * ch + h * hh, hh), :],
                    send_sem=ag_send.at[h, d],
                    recv_sem=ag_recv.at[h, d],
                    device_id=(t,),
                    device_id_type=pl.DeviceIdType.MESH,
                )
                rdma.start()
                ag_rdmas.append(rdma)

        for rdma in ag_rdmas:
            rdma.wait_recv()

        for rdma in rs_rdmas:
            rdma.wait_send()
        for rdma in ag_rdmas:
            rdma.wait_send()

    return pl.pallas_call(
        body,
        out_shape=jax.ShapeDtypeStruct((m, n), x.dtype),
        in_specs=[pl.BlockSpec(memory_space=pltpu.VMEM)],
        out_specs=pl.BlockSpec(memory_space=pltpu.VMEM),
        scratch_shapes=[
            pltpu.VMEM((H, N, hh, n), x.dtype),
            pltpu.SemaphoreType.DMA((H, N)),
            pltpu.SemaphoreType.DMA((H, N)),
            pltpu.SemaphoreType.DMA((H, N)),
            pltpu.SemaphoreType.DMA((H, N)),
        ],
    )(x)
